# baseline (speedup 1.0000x reference)
import functools

import jax
import jax.numpy as jnp
from jax import lax
from jax.experimental import pallas as pl
from jax.experimental.pallas import tpu as pltpu
from jax.experimental.pallas import tpu_sc as plsc

N, L, V, D = 16384, 200, 100000, 64
VP = 102400

_VBLK = 10240


def _scores_body(w_ref, embt_ref, t_ref):
    et = embt_ref[...]
    w = w_ref[...]
    s = lax.dot_general(w, et, (((1,), (0,)), ((), ())),
                        preferred_element_type=jnp.float32)
    t_ref[...] = jax.nn.sigmoid(s[0])


def _scores(embt, lin_w):
    return pl.pallas_call(
        _scores_body,
        grid=(VP // _VBLK,),
        in_specs=[
            pl.BlockSpec((1, D), lambda i: (0, 0)),
            pl.BlockSpec((D, _VBLK), lambda i: (0, i)),
        ],
        out_specs=pl.BlockSpec((_VBLK,), lambda i: (i,)),
        out_shape=jax.ShapeDtypeStruct((VP,), jnp.float32),
    )(lin_w.reshape(1, D), embt)


_NC, _NS = 2, 16
_NW = _NC * _NS
_RPW = N // _NW
_CBLK = 128
_NCB = _RPW // _CBLK
_LQ = 48
_LQ3 = L - 3 * _LQ


def _sc_body(t_hbm, xt_hbm, w_hbm, base_hbm, out_hbm, tb_hbm,
             t_v, xa, xb, out_v, w_v, base_v, tb_v, sem_a, sem_b):
    wid = lax.axis_index("s") * _NC + lax.axis_index("c")
    col0 = wid * _RPW

    def start_q(cb, q, buf, sem):
        nl = _LQ3 if q == 3 else _LQ
        pltpu.async_copy(
            xt_hbm.at[pl.ds(q * _LQ, nl), pl.ds(col0 + cb * _CBLK, _CBLK)],
            buf.at[pl.ds(0, nl), :], sem)

    def wait_q(q, buf, sem):
        nl = _LQ3 if q == 3 else _LQ
        pltpu.make_async_copy(
            xt_hbm.at[pl.ds(q * _LQ, nl), pl.ds(0, _CBLK)],
            buf.at[pl.ds(0, nl), :], sem).wait()

    start_q(0, 0, xa, sem_a)
    start_q(0, 1, xb, sem_b)
    with jax.named_scope("t_load"):
        pltpu.sync_copy(t_hbm.at[pl.ds(0, V)], t_v)
        pltpu.sync_copy(w_hbm, w_v)

    @pl.when(wid == 0)
    def _():
        pltpu.sync_copy(base_hbm, base_v)
        for c in range(12):
            idx = base_v[pl.ds(16 * c, 16)]
            tb_v[pl.ds(16 * c, 16)] = plsc.load_gather(t_v, [idx])
        idx = base_v[pl.ds(L - 16, 16)]
        tb_v[pl.ds(L - 16, 16)] = plsc.load_gather(t_v, [idx])
        pltpu.sync_copy(tb_v, tb_hbm)

    ngrp = _CBLK // 16

    def lanes(buf, row0, wv, ks, accs):
        accs = list(accs)
        for k in ks:
            wl = wv[k]
            for g in range(ngrp):
                idx = buf[row0 + (k - ks[0]), pl.ds(16 * g, 16)]
                val = plsc.load_gather(t_v, [idx])
                accs[g] = accs[g] + val * wl
        return tuple(accs)

    def half(buf, l0, nchunk, accs):
        def c_body(c, accs):
            lc = l0 + 8 * c
            wv = w_v[pl.ds(lc, 16)]
            return lanes(buf, 8 * c, wv, range(8), accs)

        return lax.fori_loop(0, nchunk, c_body, accs)

    def cb_body(cb, carry):
        zero = jnp.zeros((16,), jnp.float32)
        accs = tuple(zero for _ in range(ngrp))

        with jax.named_scope("wait0"):
            wait_q(0, xa, sem_a)
        accs = half(xa, 0, _LQ // 8, accs)
        start_q(cb, 2, xa, sem_a)

        with jax.named_scope("wait1"):
            wait_q(1, xb, sem_b)
        accs = half(xb, _LQ, _LQ // 8, accs)
        start_q(cb, 3, xb, sem_b)

        with jax.named_scope("wait2"):
            wait_q(2, xa, sem_a)
        accs = half(xa, 2 * _LQ, _LQ // 8, accs)

        @pl.when(cb < _NCB - 1)
        def _():
            start_q(cb + 1, 0, xa, sem_a)

        with jax.named_scope("wait3"):
            wait_q(3, xb, sem_b)
        accs = half(xb, 3 * _LQ, (_LQ3 - 8) // 8, accs)
        accs = lanes(xb, _LQ3 - 8, w_v[pl.ds(L - 16, 16)], range(8, 16), accs)

        @pl.when(cb < _NCB - 1)
        def _():
            start_q(cb + 1, 1, xb, sem_b)

        for g in range(ngrp):
            out_v[pl.ds(cb * _CBLK + 16 * g, 16)] = accs[g]
        return carry

    lax.fori_loop(0, _NCB, cb_body, 0)
    pltpu.sync_copy(out_v, out_hbm.at[pl.ds(col0, _RPW)])


_sc_call = functools.partial(
    pl.kernel,
    mesh=plsc.VectorSubcoreMesh(core_axis_name="c", subcore_axis_name="s"),
    out_type=[jax.ShapeDtypeStruct((N,), jnp.float32),
              jax.ShapeDtypeStruct((L,), jnp.float32)],
    compiler_params=pltpu.CompilerParams(needs_layout_passes=False),
    scratch_types=[
        pltpu.VMEM((V,), jnp.float32),
        pltpu.VMEM((_LQ3, _CBLK), jnp.int32),
        pltpu.VMEM((_LQ3, _CBLK), jnp.int32),
        pltpu.VMEM((_RPW,), jnp.float32),
        pltpu.VMEM((L,), jnp.float32),
        pltpu.VMEM((L,), jnp.int32),
        pltpu.VMEM((L,), jnp.float32),
        pltpu.SemaphoreType.DMA,
        pltpu.SemaphoreType.DMA,
    ],
)(_sc_body)


_CB = 1024


def _combine_body(xt_ref, base_ref, w_ref, b_ref, tb_ref, acc_ref, out_ref):
    xt = xt_ref[...]
    c = tb_ref[...] * w_ref[...] + b_ref[...]
    maskf = (xt == base_ref[...]).astype(jnp.float32)
    corr = lax.dot_general(c, maskf, (((0,), (0,)), ((), ())),
                           preferred_element_type=jnp.float32)
    bsum = jnp.sum(b_ref[...])
    out_ref[...] = acc_ref[...] + (bsum - corr[0])


def _combine(xt, base2, w2, b2, tb2, acc):
    return pl.pallas_call(
        _combine_body,
        grid=(N // _CB,),
        in_specs=[
            pl.BlockSpec((L, _CB), lambda i: (0, i)),
            pl.BlockSpec((L, 1), lambda i: (0, 0)),
            pl.BlockSpec((L, 1), lambda i: (0, 0)),
            pl.BlockSpec((L, 1), lambda i: (0, 0)),
            pl.BlockSpec((L, 1), lambda i: (0, 0)),
            pl.BlockSpec((_CB,), lambda i: (i,)),
        ],
        out_specs=pl.BlockSpec((_CB,), lambda i: (i,)),
        out_shape=jax.ShapeDtypeStruct((N,), jnp.float32),
    )(xt, base2, w2, b2, tb2, acc)


def kernel(x, emb, lin_w, weight_layer, bias_layer, base_seq):
    xt = x.astype(jnp.int32).T
    t = _scores(emb.astype(jnp.float32).T, lin_w.astype(jnp.float32))
    w = weight_layer.astype(jnp.float32)
    b = bias_layer.astype(jnp.float32)
    s = base_seq.astype(jnp.int32)
    acc, tb = _sc_call(t, xt, w, s)
    return _combine(xt, s.reshape(L, 1), w.reshape(L, 1), b.reshape(L, 1),
                    tb.reshape(L, 1), acc)


# speedup vs baseline: 1.0902x; 1.0902x over previous
import functools

import jax
import jax.numpy as jnp
from jax import lax
from jax.experimental import pallas as pl
from jax.experimental.pallas import tpu as pltpu
from jax.experimental.pallas import tpu_sc as plsc

N, L, V, D = 16384, 200, 100000, 64
VP = 102400

_VBLK = 10240


def _scores_body(w_ref, embt_ref, t_ref):
    et = embt_ref[...]
    w = w_ref[...]
    s = lax.dot_general(w, et, (((1,), (0,)), ((), ())),
                        preferred_element_type=jnp.float32)
    t_ref[...] = jax.nn.sigmoid(s[0])


def _scores(embt, lin_w):
    return pl.pallas_call(
        _scores_body,
        grid=(VP // _VBLK,),
        in_specs=[
            pl.BlockSpec((1, D), lambda i: (0, 0)),
            pl.BlockSpec((D, _VBLK), lambda i: (0, i)),
        ],
        out_specs=pl.BlockSpec((_VBLK,), lambda i: (i,)),
        out_shape=jax.ShapeDtypeStruct((VP,), jnp.float32),
    )(lin_w.reshape(1, D), embt)


_NC, _NS = 2, 16
_NW = _NC * _NS
_RPW = N // _NW
_CBLK = 128
_NCB = _RPW // _CBLK
_LQ = 48
_LQ3 = L - 3 * _LQ


def _sc_body(t_hbm, xt_hbm, w_hbm, base_hbm, out_hbm, tb_hbm,
             t_v, xa, xb, out_v, w_v, base_v, tb_v, sem_a, sem_b):
    wid = lax.axis_index("s") * _NC + lax.axis_index("c")
    col0 = wid * _RPW

    def start_q(cb, q, buf, sem):
        nl = _LQ3 if q == 3 else _LQ
        pltpu.async_copy(
            xt_hbm.at[pl.ds(q * _LQ, nl), pl.ds(col0 + cb * _CBLK, _CBLK)],
            buf.at[pl.ds(0, nl), :], sem)

    def wait_q(q, buf, sem):
        nl = _LQ3 if q == 3 else _LQ
        pltpu.make_async_copy(
            xt_hbm.at[pl.ds(q * _LQ, nl), pl.ds(0, _CBLK)],
            buf.at[pl.ds(0, nl), :], sem).wait()

    start_q(0, 0, xa, sem_a)
    start_q(0, 1, xb, sem_b)
    with jax.named_scope("t_load"):
        pltpu.sync_copy(t_hbm.at[pl.ds(0, V)], t_v)
        pltpu.sync_copy(w_hbm, w_v)

    @pl.when(wid == 0)
    def _():
        pltpu.sync_copy(base_hbm, base_v)
        for c in range(12):
            idx = base_v[pl.ds(16 * c, 16)]
            tb_v[pl.ds(16 * c, 16)] = plsc.load_gather(t_v, [idx])
        idx = base_v[pl.ds(L - 16, 16)]
        tb_v[pl.ds(L - 16, 16)] = plsc.load_gather(t_v, [idx])
        pltpu.sync_copy(tb_v, tb_hbm)

    ngrp = _CBLK // 16

    def lanes(buf, row0, wv, ks, accs):
        accs = list(accs)
        for k in ks:
            wl = wv[k]
            for g in range(ngrp):
                idx = buf[row0 + (k - ks[0]), pl.ds(16 * g, 16)]
                val = plsc.load_gather(t_v, [idx])
                accs[g] = accs[g] + val * wl
        return tuple(accs)

    def half(buf, l0, nchunk, accs):
        def c_body(c, accs):
            lc = l0 + 8 * c
            wv = w_v[pl.ds(lc, 16)]
            return lanes(buf, 8 * c, wv, range(8), accs)

        return lax.fori_loop(0, nchunk, c_body, accs)

    def cb_body(cb, carry):
        zero = jnp.zeros((16,), jnp.float32)
        accs = tuple(zero for _ in range(ngrp))

        with jax.named_scope("wait0"):
            wait_q(0, xa, sem_a)
        accs = half(xa, 0, _LQ // 8, accs)
        start_q(cb, 2, xa, sem_a)

        with jax.named_scope("wait1"):
            wait_q(1, xb, sem_b)
        accs = half(xb, _LQ, _LQ // 8, accs)
        start_q(cb, 3, xb, sem_b)

        with jax.named_scope("wait2"):
            wait_q(2, xa, sem_a)
        accs = half(xa, 2 * _LQ, _LQ // 8, accs)

        @pl.when(cb < _NCB - 1)
        def _():
            start_q(cb + 1, 0, xa, sem_a)

        with jax.named_scope("wait3"):
            wait_q(3, xb, sem_b)
        accs = half(xb, 3 * _LQ, (_LQ3 - 8) // 8, accs)
        accs = lanes(xb, _LQ3 - 8, w_v[pl.ds(L - 16, 16)], range(8, 16), accs)

        @pl.when(cb < _NCB - 1)
        def _():
            start_q(cb + 1, 1, xb, sem_b)

        for g in range(ngrp):
            out_v[pl.ds(cb * _CBLK + 16 * g, 16)] = accs[g]
        return carry

    lax.fori_loop(0, _NCB, cb_body, 0)
    pltpu.sync_copy(out_v, out_hbm.at[pl.ds(col0, _RPW)])


_sc_call = functools.partial(
    pl.kernel,
    mesh=plsc.VectorSubcoreMesh(core_axis_name="c", subcore_axis_name="s"),
    out_type=[jax.ShapeDtypeStruct((N,), jnp.float32),
              jax.ShapeDtypeStruct((L,), jnp.float32)],
    compiler_params=pltpu.CompilerParams(needs_layout_passes=False),
    scratch_types=[
        pltpu.VMEM((V,), jnp.float32),
        pltpu.VMEM((_LQ3, _CBLK), jnp.int32),
        pltpu.VMEM((_LQ3, _CBLK), jnp.int32),
        pltpu.VMEM((_RPW,), jnp.float32),
        pltpu.VMEM((L,), jnp.float32),
        pltpu.VMEM((L,), jnp.int32),
        pltpu.VMEM((L,), jnp.float32),
        pltpu.SemaphoreType.DMA,
        pltpu.SemaphoreType.DMA,
    ],
)(_sc_body)


_CB = 4096


def _combine_body(xt_ref, base_ref, w_ref, b_ref, tb_ref, acc_ref, out_ref):
    xt = xt_ref[...]
    c = tb_ref[...] * w_ref[...] + b_ref[...]
    maskf = (xt == base_ref[...]).astype(jnp.float32)
    corr = lax.dot_general(c, maskf, (((0,), (0,)), ((), ())),
                           preferred_element_type=jnp.float32)
    bsum = jnp.sum(b_ref[...])
    out_ref[...] = acc_ref[...] + (bsum - corr[0])


def _combine(xt, base2, w2, b2, tb2, acc):
    return pl.pallas_call(
        _combine_body,
        grid=(N // _CB,),
        in_specs=[
            pl.BlockSpec((L, _CB), lambda i: (0, i)),
            pl.BlockSpec((L, 1), lambda i: (0, 0)),
            pl.BlockSpec((L, 1), lambda i: (0, 0)),
            pl.BlockSpec((L, 1), lambda i: (0, 0)),
            pl.BlockSpec((L, 1), lambda i: (0, 0)),
            pl.BlockSpec((_CB,), lambda i: (i,)),
        ],
        out_specs=pl.BlockSpec((_CB,), lambda i: (i,)),
        out_shape=jax.ShapeDtypeStruct((N,), jnp.float32),
    )(xt, base2, w2, b2, tb2, acc)


def kernel(x, emb, lin_w, weight_layer, bias_layer, base_seq):
    xt = x.astype(jnp.int32).T
    t = _scores(emb.astype(jnp.float32).T, lin_w.astype(jnp.float32))
    w = weight_layer.astype(jnp.float32)
    b = bias_layer.astype(jnp.float32)
    s = base_seq.astype(jnp.int32)
    acc, tb = _sc_call(t, xt, w, s)
    return _combine(xt, s.reshape(L, 1), w.reshape(L, 1), b.reshape(L, 1),
                    tb.reshape(L, 1), acc)
